# Initial kernel scaffold; baseline (speedup 1.0000x reference)
import jax
import jax.numpy as jnp
from jax.experimental import pallas as pl


def kernel(x, edge_index, W1a, b1a, W1b, b1b, W2a, b2a, W2b, b2b):
    raise NotImplementedError("write your pallas kernel here")


# speedup vs baseline: 6.4433x; 6.4433x over previous
import functools

import jax
import jax.numpy as jnp
from jax import lax
from jax.experimental import pallas as pl
from jax.experimental.pallas import tpu as pltpu
from jax.experimental.pallas import tpu_sc as plsc

N_NODES = 10000
N_EDGES = 320000
D = 128
D_HID = 16
CH = 128
NCHUNK = N_EDGES // CH
NC, NS = 2, 16
CPC = NCHUNK // NC
MAXC = CPC // NS + 1
RPT = 624
TAIL0 = NS * RPT
TAILN = N_NODES - TAIL0


def _sc_agg_body(x_hbm, src_hbm, dst_hbm, zeros_hbm, out_hbm,
                 src_idx, dst_idx, rows, agg, gsem):
    c = lax.axis_index("c")
    s = lax.axis_index("s")
    r0 = s * RPT

    @pl.when(c == 0)
    def _():
        pltpu.sync_copy(x_hbm.at[pl.ds(r0, RPT)], agg.at[pl.ds(r0, RPT)])

        @pl.when(s == NS - 1)
        def _():
            pltpu.sync_copy(x_hbm.at[pl.ds(TAIL0, TAILN)],
                            agg.at[pl.ds(TAIL0, TAILN)])

    @pl.when(c != 0)
    def _():
        pltpu.sync_copy(zeros_hbm.at[pl.ds(r0, RPT)], agg.at[pl.ds(r0, RPT)])

        @pl.when(s == NS - 1)
        def _():
            pltpu.sync_copy(zeros_hbm.at[pl.ds(TAIL0, TAILN)],
                            agg.at[pl.ds(TAIL0, TAILN)])

    start = (s * CPC) // NS
    cnt = ((s + 1) * CPC) // NS - start
    g0 = c * CPC + start

    plsc.subcore_barrier()

    def body(j, carry):
        @pl.when(j < cnt)
        def _():
            e0 = (g0 + j) * CH
            pltpu.sync_copy(src_hbm.at[pl.ds(e0, CH)], src_idx)
            pltpu.sync_copy(dst_hbm.at[pl.ds(e0, CH)], dst_idx)
            pltpu.async_copy(x_hbm.at[src_idx], rows, gsem).wait()
            pltpu.sync_copy(rows, agg.at[dst_idx], add=True)
        return carry

    lax.fori_loop(0, MAXC, body, 0)

    plsc.subcore_barrier()
    pltpu.sync_copy(agg.at[pl.ds(r0, RPT)], out_hbm.at[c, pl.ds(r0, RPT)])

    @pl.when(s == NS - 1)
    def _():
        pltpu.sync_copy(agg.at[pl.ds(TAIL0, TAILN)],
                        out_hbm.at[c, pl.ds(TAIL0, TAILN)])


_sc_agg = pl.kernel(
    _sc_agg_body,
    out_type=jax.ShapeDtypeStruct((NC, N_NODES, D), jnp.float32),
    mesh=plsc.VectorSubcoreMesh(
        core_axis_name="c", subcore_axis_name="s",
        num_cores=NC, num_subcores=NS),
    scratch_types=[
        pltpu.VMEM((CH,), jnp.int32),
        pltpu.VMEM((CH,), jnp.int32),
        pltpu.VMEM((CH, D), jnp.float32),
        pltpu.VMEM_SHARED((N_NODES, D), jnp.float32),
        pltpu.SemaphoreType.DMA,
    ],
)


def _mlp_body(p_ref, wa_ref, ba_ref, wb_ref, bb_ref, o_ref, *, relu_out):
    h = p_ref[0] + p_ref[1]
    t = jnp.dot(h, wa_ref[...], preferred_element_type=jnp.float32)
    t = jnp.maximum(t + ba_ref[...], 0.0)
    y = jnp.dot(t, wb_ref[...], preferred_element_type=jnp.float32)
    y = y + bb_ref[...]
    if relu_out:
        y = jnp.maximum(y, 0.0)
    o_ref[...] = y


def _mlp(p, wa, ba, wb, bb, relu_out):
    B = 2000
    return pl.pallas_call(
        functools.partial(_mlp_body, relu_out=relu_out),
        grid=(N_NODES // B,),
        in_specs=[
            pl.BlockSpec((NC, B, D), lambda i: (0, i, 0)),
            pl.BlockSpec((D, D_HID), lambda i: (0, 0)),
            pl.BlockSpec((1, D_HID), lambda i: (0, 0)),
            pl.BlockSpec((D_HID, D), lambda i: (0, 0)),
            pl.BlockSpec((1, D), lambda i: (0, 0)),
        ],
        out_specs=pl.BlockSpec((B, D), lambda i: (i, 0)),
        out_shape=jax.ShapeDtypeStruct((N_NODES, D), jnp.float32),
    )(p, wa, ba.reshape(1, D_HID), wb, bb.reshape(1, D))


def kernel(x, edge_index, W1a, b1a, W1b, b1b, W2a, b2a, W2b, b2b):
    ei = edge_index.astype(jnp.int32)
    src = ei[0]
    dst = ei[1]
    zeros = jnp.zeros((N_NODES, D), jnp.float32)
    p1 = _sc_agg(x, src, dst, zeros)
    h = _mlp(p1, W1a, b1a, W1b, b1b, True)
    p2 = _sc_agg(h, src, dst, zeros)
    return _mlp(p2, W2a, b2a, W2b, b2b, False)
